# Initial kernel scaffold; baseline (speedup 1.0000x reference)
import functools

import jax
import jax.numpy as jnp
from jax import lax
from jax.experimental import pallas as pl
from jax.experimental.pallas import tpu as pltpu
from jax.experimental.pallas import tpu_sc as plsc

_L = 10000
_D = 128
_E = 320000
_NC = 2
_NS = 16
_NW = _NC * _NS
_C = 128
_NCH = 79
_EPT = _C * _NCH
_EPAD = _NW * _EPT
_LP = 10240
_RPT = _LP // _NS
_DW = 16

_mesh = plsc.VectorSubcoreMesh(core_axis_name="c", subcore_axis_name="s")


@functools.partial(
    pl.kernel,
    mesh=_mesh,
    out_type=jax.ShapeDtypeStruct((_NC, _LP, _DW), jnp.float32),
    scratch_types=[
        pltpu.VMEM((_NCH, _C), jnp.int32),
        pltpu.VMEM((_C, _DW), jnp.float32),
        pltpu.VMEM((_RPT, _DW), jnp.float32),
        pltpu.VMEM_SHARED((_LP, _DW), jnp.float32),
    ],
)
def _deg_sc(dst_hbm, zeros_hbm, ones_hbm, out_hbm, dst_v, ones_v, stage_v, acc):
    cid = lax.axis_index("c")
    sid = lax.axis_index("s")
    wid = sid * _NC + cid
    r0 = sid * _RPT
    pltpu.sync_copy(zeros_hbm.at[pl.ds(r0, _RPT)], acc.at[pl.ds(r0, _RPT)])
    pltpu.sync_copy(ones_hbm, ones_v)
    pltpu.sync_copy(dst_hbm.at[wid], dst_v)
    plsc.subcore_barrier()

    def body(j, carry):
        pltpu.sync_copy(ones_v, acc.at[dst_v.at[j]], add=True)
        return carry

    lax.fori_loop(0, _NCH, body, 0)
    plsc.subcore_barrier()
    pltpu.sync_copy(acc.at[pl.ds(r0, _RPT)], stage_v)
    pltpu.sync_copy(stage_v, out_hbm.at[cid, pl.ds(r0, _RPT)])


@functools.partial(
    pl.kernel,
    mesh=_mesh,
    out_type=jax.ShapeDtypeStruct((_NC, _LP, _D), jnp.float32),
    scratch_types=[
        pltpu.VMEM((_NCH, _C), jnp.int32),
        pltpu.VMEM((_NCH, _C), jnp.int32),
        pltpu.VMEM((_C, _D), jnp.float32),
        pltpu.VMEM_SHARED((_LP, _D), jnp.float32),
        pltpu.SemaphoreType.DMA,
    ],
)
def _segsum_sc(h_hbm, src_hbm, dst_hbm, zeros_hbm, out_hbm,
               src_v, dst_v, rows_v, acc, sem):
    cid = lax.axis_index("c")
    sid = lax.axis_index("s")
    wid = sid * _NC + cid
    r0 = sid * _RPT
    pltpu.sync_copy(zeros_hbm.at[pl.ds(r0, _RPT)], acc.at[pl.ds(r0, _RPT)])
    pltpu.sync_copy(src_hbm.at[wid], src_v)
    pltpu.sync_copy(dst_hbm.at[wid], dst_v)
    plsc.subcore_barrier()

    def body(j, carry):
        pltpu.async_copy(h_hbm.at[src_v.at[j]], rows_v, sem).wait()
        pltpu.sync_copy(rows_v, acc.at[dst_v.at[j]], add=True)
        return carry

    lax.fori_loop(0, _NCH, body, 0)
    plsc.subcore_barrier()

    def wb(k, carry):
        rr = r0 + k * _C
        pltpu.sync_copy(acc.at[pl.ds(rr, _C)], rows_v)
        pltpu.sync_copy(rows_v, out_hbm.at[cid, pl.ds(rr, _C)])
        return carry

    lax.fori_loop(0, _RPT // _C, wb, 0)


def _proj_body(x_ref, w_ref, b_ref, o_ref):
    o_ref[...] = lax.dot_general(
        x_ref[...], w_ref[...], (((1,), (1,)), ((), ())),
        preferred_element_type=jnp.float32,
        precision=lax.Precision.HIGHEST) + b_ref[...]


_proj = pl.pallas_call(
    _proj_body, out_shape=jax.ShapeDtypeStruct((_L, _D), jnp.float32))


def _layer_body(h_ref, p0_ref, p1_ref, dp_ref, w1_ref, b1_ref, w2_ref, b2_ref,
                g_ref, bb_ref, o_ref):
    deg = dp_ref[0, :_L] + dp_ref[1, :_L]
    inv = 1.0 / jnp.maximum(deg, 1.0)
    m = (p0_ref[...] + p1_ref[...]) * inv[:, None]
    z = lax.dot_general(
        m, w1_ref[...], (((1,), (1,)), ((), ())),
        preferred_element_type=jnp.float32,
        precision=lax.Precision.HIGHEST) + b1_ref[...]
    z = jnp.maximum(z, 0.0)
    z = lax.dot_general(
        z, w2_ref[...], (((1,), (1,)), ((), ())),
        preferred_element_type=jnp.float32,
        precision=lax.Precision.HIGHEST) + b2_ref[...]
    v = h_ref[...] + z
    mu = jnp.mean(v, axis=-1, keepdims=True)
    var = jnp.mean((v - mu) ** 2, axis=-1, keepdims=True)
    o_ref[...] = (v - mu) * lax.rsqrt(var + 1e-5) * g_ref[...] + bb_ref[...]


_layer = pl.pallas_call(
    _layer_body, out_shape=jax.ShapeDtypeStruct((_L, _D), jnp.float32))


def kernel(x, edge_index, in_w, in_b, w1, b1, w2, b2, ln_g, ln_b):
    src = edge_index[0]
    dst = edge_index[1]
    pad = _EPAD - _E
    srcp = jnp.concatenate([src, jnp.zeros((pad,), jnp.int32)]).reshape(
        _NW, _NCH, _C)
    dstp = jnp.concatenate([dst, jnp.full((pad,), _L, jnp.int32)]).reshape(
        _NW, _NCH, _C)
    zeros_nd = jnp.zeros((_LP, _D), jnp.float32)
    zeros_dw = jnp.zeros((_LP, _DW), jnp.float32)
    ones_dw = jnp.ones((_C, _DW), jnp.float32)

    degp = _deg_sc(dstp, zeros_dw, ones_dw)
    dp = degp[:, :, 0]

    h = _proj(x, in_w, in_b[None, :])
    for i in range(w1.shape[0]):
        parts = _segsum_sc(h, srcp, dstp, zeros_nd)
        h = _layer(h, parts[0, :_L], parts[1, :_L], dp,
                   w1[i], b1[i][None, :], w2[i], b2[i][None, :],
                   ln_g[i][None, :], ln_b[i][None, :])
    return h


# speedup vs baseline: 3.4194x; 3.4194x over previous
import functools

import jax
import jax.numpy as jnp
from jax import lax
from jax.experimental import pallas as pl
from jax.experimental.pallas import tpu as pltpu
from jax.experimental.pallas import tpu_sc as plsc

_L = 10000
_D = 128
_E = 320000
_NC = 2
_NS = 16
_NW = _NC * _NS
_C = 128
_NCH = 79
_EPT = _C * _NCH
_EPAD = _NW * _EPT
_LP = 10240
_RPT = _LP // _NS

_mesh = plsc.VectorSubcoreMesh(core_axis_name="c", subcore_axis_name="s")


def _make_sc(gather):
    @functools.partial(
        pl.kernel,
        mesh=_mesh,
        out_type=jax.ShapeDtypeStruct((_NC * _LP, _D), jnp.float32),
        scratch_types=[
            pltpu.VMEM((_C,), jnp.int32),
            pltpu.VMEM((_C,), jnp.int32),
            pltpu.VMEM((_C, _D), jnp.float32),
            pltpu.VMEM_SHARED((_LP, _D), jnp.float32),
            pltpu.SemaphoreType.DMA,
        ],
    )
    def k(tbl_hbm, src_hbm, dst_hbm, zeros_hbm, out_hbm,
          idx_s, idx_d, rows_v, acc, sem):
        cid = lax.axis_index("c")
        sid = lax.axis_index("s")
        wid = sid * _NC + cid
        r0 = sid * _RPT
        pltpu.sync_copy(zeros_hbm.at[pl.ds(r0, _RPT)], acc.at[pl.ds(r0, _RPT)])
        if not gather:
            pltpu.sync_copy(tbl_hbm.at[pl.ds(0, _C)], rows_v)
        plsc.subcore_barrier()

        def body(j, carry):
            if gather:
                pltpu.sync_copy(src_hbm.at[wid * _NCH + j], idx_s)
            pltpu.sync_copy(dst_hbm.at[wid * _NCH + j], idx_d)
            if gather:
                pltpu.async_copy(tbl_hbm.at[idx_s], rows_v, sem).wait()
            pltpu.sync_copy(rows_v, acc.at[idx_d], add=True)
            return carry

        lax.fori_loop(0, _NCH, body, 0)
        plsc.subcore_barrier()

        def wb(kk, carry):
            rr = r0 + kk * _C
            pltpu.sync_copy(acc.at[pl.ds(rr, _C)], rows_v)
            pltpu.sync_copy(rows_v, out_hbm.at[pl.ds(cid * _LP + rr, _C)])
            return carry

        lax.fori_loop(0, _RPT // _C, wb, 0)

    return k


_segsum_sc = _make_sc(gather=True)
_deg_sc = _make_sc(gather=False)


def _proj_body(x_ref, w_ref, b_ref, o_ref):
    o_ref[...] = lax.dot_general(
        x_ref[...], w_ref[...], (((1,), (1,)), ((), ())),
        preferred_element_type=jnp.float32,
        precision=lax.Precision.HIGHEST) + b_ref[...]


_proj = pl.pallas_call(
    _proj_body, out_shape=jax.ShapeDtypeStruct((_L, _D), jnp.float32))


def _layer_body(h_ref, p0_ref, p1_ref, dp_ref, w1_ref, b1_ref, w2_ref, b2_ref,
                g_ref, bb_ref, o_ref):
    deg = dp_ref[0, :_L] + dp_ref[1, :_L]
    inv = 1.0 / jnp.maximum(deg, 1.0)
    m = (p0_ref[...] + p1_ref[...]) * inv[:, None]
    z = lax.dot_general(
        m, w1_ref[...], (((1,), (1,)), ((), ())),
        preferred_element_type=jnp.float32,
        precision=lax.Precision.HIGHEST) + b1_ref[...]
    z = jnp.maximum(z, 0.0)
    z = lax.dot_general(
        z, w2_ref[...], (((1,), (1,)), ((), ())),
        preferred_element_type=jnp.float32,
        precision=lax.Precision.HIGHEST) + b2_ref[...]
    v = h_ref[...] + z
    mu = jnp.mean(v, axis=-1, keepdims=True)
    var = jnp.mean((v - mu) ** 2, axis=-1, keepdims=True)
    o_ref[...] = (v - mu) * lax.rsqrt(var + 1e-5) * g_ref[...] + bb_ref[...]


_layer = pl.pallas_call(
    _layer_body, out_shape=jax.ShapeDtypeStruct((_L, _D), jnp.float32))


def kernel(x, edge_index, in_w, in_b, w1, b1, w2, b2, ln_g, ln_b):
    src = edge_index[0]
    dst = edge_index[1]
    pad = _EPAD - _E
    srcp = jnp.concatenate([src, jnp.zeros((pad,), jnp.int32)]).reshape(
        _NW * _NCH, _C)
    dstp = jnp.concatenate([dst, jnp.full((pad,), _L, jnp.int32)]).reshape(
        _NW * _NCH, _C)
    zeros_nd = jnp.zeros((_LP, _D), jnp.float32)
    ones_nd = jnp.ones((_C, _D), jnp.float32)

    degp = _deg_sc(ones_nd, srcp, dstp, zeros_nd)
    dp = degp[:, 0].reshape(_NC, _LP)

    h = _proj(x, in_w, in_b[None, :])
    for i in range(w1.shape[0]):
        parts = _segsum_sc(h, srcp, dstp, zeros_nd)
        h = _layer(h, parts[:_L], parts[_LP:_LP + _L], dp,
                   w1[i], b1[i][None, :], w2[i], b2[i][None, :],
                   ln_g[i][None, :], ln_b[i][None, :])
    return h
